# baseline (speedup 1.0000x reference)
import functools

import jax
import jax.numpy as jnp
from jax import lax
from jax.experimental import pallas as pl
from jax.experimental.pallas import tpu as pltpu
from jax.experimental.pallas import tpu_sc as plsc

_N = 10000
_C = 64
_HID = 64
_F_IN = 128
_K = 10
_ALPHA = 0.1

_N_TILES = 32
_N_PAD = 10240
_ROWS_PER_TILE = _N_PAD // 16
_CH = 128


_SLAB = 4


def _make_edge_sweep(n_slabs):
  mesh = plsc.VectorSubcoreMesh(core_axis_name="c", subcore_axis_name="s")

  @functools.partial(
      pl.kernel,
      mesh=mesh,
      compiler_params=pltpu.CompilerParams(use_tc_tiling_on_sc=False),
      out_type=jax.ShapeDtypeStruct((2, _N_PAD, _C), jnp.float32),
      scratch_types=[
          pltpu.VMEM_SHARED((_N_PAD, _C), jnp.float32),
          pltpu.VMEM((n_slabs, _SLAB * _CH), jnp.int32),
          pltpu.VMEM((n_slabs, _SLAB * _CH), jnp.int32),
          pltpu.VMEM((_SLAB * _CH, _C), jnp.float32),
          pltpu.VMEM((_CH, _C), jnp.float32),
          pltpu.SemaphoreType.DMA,
      ],
  )
  def sweep(u_hbm, src_hbm, dst_hbm, zeros_hbm, agg_hbm,
            acc_sh, sidx_v, didx_v, rows_v, zbuf_v, sem):
    cid = lax.axis_index("c")
    sid = lax.axis_index("s")
    wid = cid * 16 + sid
    row0 = sid * _ROWS_PER_TILE

    pltpu.sync_copy(zeros_hbm, zbuf_v)
    for b in range(_ROWS_PER_TILE // _CH):
      pltpu.sync_copy(zbuf_v, acc_sh.at[pl.ds(row0 + b * _CH, _CH)])
    pltpu.sync_copy(src_hbm.at[wid], sidx_v)
    pltpu.sync_copy(dst_hbm.at[wid], didx_v)
    plsc.subcore_barrier()

    def body(j, carry):
      pltpu.async_copy(u_hbm.at[sidx_v.at[j]], rows_v, sem).wait()
      pltpu.sync_copy(rows_v, acc_sh.at[didx_v.at[j]], add=True)
      return carry

    lax.fori_loop(0, n_slabs, body, 0)
    plsc.subcore_barrier()

    for b in range(_ROWS_PER_TILE // _CH):
      pltpu.sync_copy(acc_sh.at[pl.ds(row0 + b * _CH, _CH)], zbuf_v)
      pltpu.sync_copy(zbuf_v, agg_hbm.at[cid, pl.ds(row0 + b * _CH, _CH)])

  return sweep


def _prep_body(x_ref, w1_ref, b1_ref, w2_ref, b2_ref, dega_ref,
               u0_ref, c_ref, bh_ref, sd_ref):
  deg = dega_ref[0, :, 0:1] + dega_ref[1, :, 0:1]
  rows = lax.broadcasted_iota(jnp.int32, (_N_PAD, 1), 0)
  mask = rows < _N
  dinv = jnp.where(mask, lax.rsqrt(jnp.maximum(deg, 1e-12)), 0.0)
  h = jax.nn.relu(
      jnp.dot(x_ref[...], w1_ref[...], preferred_element_type=jnp.float32)
      + b1_ref[...])
  h = jnp.dot(h, w2_ref[...], preferred_element_type=jnp.float32) + b2_ref[...]
  u0_ref[...] = jnp.broadcast_to(dinv, (_N_PAD, _C)) * h
  c_ref[...] = jnp.broadcast_to((1.0 - _ALPHA) * dinv * dinv, (_N_PAD, _C))
  bh_ref[...] = _ALPHA * jnp.broadcast_to(dinv, (_N_PAD, _C)) * h
  sd_ref[...] = jnp.broadcast_to(
      jnp.where(mask, 1.0 / jnp.where(mask, dinv, 1.0), 0.0), (_N_PAD, _C))


_prep_call = pl.pallas_call(
    _prep_body,
    out_shape=[jax.ShapeDtypeStruct((_N_PAD, _C), jnp.float32)] * 4,
)


def _update_body(agg_ref, c_ref, bh_ref, u_ref):
  u_ref[...] = c_ref[...] * (agg_ref[0] + agg_ref[1]) + bh_ref[...]


_update_call = pl.pallas_call(
    _update_body,
    out_shape=jax.ShapeDtypeStruct((_N_PAD, _C), jnp.float32),
)


def _final_body(u_ref, sd_ref, out_ref):
  z = (u_ref[...] * sd_ref[...])[:_N]
  m = jnp.max(z, axis=1, keepdims=True)
  shifted = z - m
  out_ref[...] = shifted - jnp.log(
      jnp.sum(jnp.exp(shifted), axis=1, keepdims=True))


_final_call = pl.pallas_call(
    _final_body,
    out_shape=jax.ShapeDtypeStruct((_N, _C), jnp.float32),
)


def kernel(x, edge_index, W1, b1, W2, b2):
  e = edge_index.shape[1]
  e_full = e + _N
  n_slabs = -(-e_full // (_N_TILES * _SLAB * _CH))
  e_pad = _N_TILES * n_slabs * _SLAB * _CH

  src = edge_index[0]
  dst = edge_index[1]
  loop = jnp.arange(_N, dtype=jnp.int32)
  pad = jnp.full((e_pad - e_full,), _N, dtype=jnp.int32)
  src_w = jnp.concatenate([src, loop, pad]).reshape(
      _N_TILES, n_slabs, _SLAB * _CH)
  dst_w = jnp.concatenate([dst, loop, pad]).reshape(
      _N_TILES, n_slabs, _SLAB * _CH)

  x_pad = jnp.pad(x, ((0, _N_PAD - _N), (0, 0)))
  zeros_tile = jnp.zeros((_CH, _C), jnp.float32)
  ones_u = jnp.ones((_N_PAD, _C), jnp.float32)

  sweep = _make_edge_sweep(n_slabs)

  dega = sweep(ones_u, src_w, dst_w, zeros_tile)
  u, c, bh, sd = _prep_call(x_pad, W1, b1.reshape(1, _HID), W2,
                            b2.reshape(1, _C), dega)
  for _ in range(_K):
    agg = sweep(u, src_w, dst_w, zeros_tile)
    u = _update_call(agg, c, bh)
  return _final_call(u, sd)


# speedup vs baseline: 3.0392x; 3.0392x over previous
import functools

import jax
import jax.numpy as jnp
from jax import lax
from jax.experimental import pallas as pl
from jax.experimental.pallas import tpu as pltpu
from jax.experimental.pallas import tpu_sc as plsc

_N = 10000
_C = 64
_HID = 64
_F_IN = 128
_K = 10
_ALPHA = 0.1

_N_TILES = 32
_N_PAD = 10240
_ROWS_PER_TILE = _N_PAD // 16
_CH = 128


_SLAB = 1


def _make_edge_sweep(n_slabs):
  mesh = plsc.VectorSubcoreMesh(core_axis_name="c", subcore_axis_name="s")

  @functools.partial(
      pl.kernel,
      mesh=mesh,
      compiler_params=pltpu.CompilerParams(use_tc_tiling_on_sc=False),
      out_type=jax.ShapeDtypeStruct((2, _N_PAD, _C), jnp.float32),
      scratch_types=[
          pltpu.VMEM_SHARED((_N_PAD, _C), jnp.float32),
          pltpu.VMEM_SHARED((_N_PAD, _C), jnp.float32),
          pltpu.VMEM((n_slabs, _SLAB * _CH), jnp.int32),
          pltpu.VMEM((n_slabs, _SLAB * _CH), jnp.int32),
          pltpu.VMEM((_SLAB * _CH, _C), jnp.float32),
          pltpu.VMEM((_CH, _C), jnp.float32),
          pltpu.SemaphoreType.DMA,
      ],
  )
  def sweep(u_hbm, src_hbm, dst_hbm, zeros_hbm, agg_hbm,
            acc_sh, u_sh, sidx_v, didx_v, rows_v, zbuf_v, sem):
    cid = lax.axis_index("c")
    sid = lax.axis_index("s")
    wid = cid * 16 + sid
    row0 = sid * _ROWS_PER_TILE

    pltpu.sync_copy(zeros_hbm, zbuf_v)
    for b in range(_ROWS_PER_TILE // _CH):
      pltpu.sync_copy(zbuf_v, acc_sh.at[pl.ds(row0 + b * _CH, _CH)])
    pltpu.sync_copy(u_hbm.at[pl.ds(row0, _ROWS_PER_TILE)],
                    u_sh.at[pl.ds(row0, _ROWS_PER_TILE)])
    pltpu.sync_copy(src_hbm.at[wid], sidx_v)
    pltpu.sync_copy(dst_hbm.at[wid], didx_v)
    plsc.subcore_barrier()

    def body(j, carry):
      pltpu.async_copy(u_sh.at[sidx_v.at[j]], rows_v, sem).wait()
      pltpu.sync_copy(rows_v, acc_sh.at[didx_v.at[j]], add=True)
      return carry

    lax.fori_loop(0, n_slabs, body, 0)
    plsc.subcore_barrier()

    for b in range(_ROWS_PER_TILE // _CH):
      pltpu.sync_copy(acc_sh.at[pl.ds(row0 + b * _CH, _CH)], zbuf_v)
      pltpu.sync_copy(zbuf_v, agg_hbm.at[cid, pl.ds(row0 + b * _CH, _CH)])

  return sweep


def _prep_body(x_ref, w1_ref, b1_ref, w2_ref, b2_ref, dega_ref,
               u0_ref, c_ref, bh_ref, sd_ref):
  deg = dega_ref[0, :, 0:1] + dega_ref[1, :, 0:1]
  rows = lax.broadcasted_iota(jnp.int32, (_N_PAD, 1), 0)
  mask = rows < _N
  dinv = jnp.where(mask, lax.rsqrt(jnp.maximum(deg, 1e-12)), 0.0)
  h = jax.nn.relu(
      jnp.dot(x_ref[...], w1_ref[...], preferred_element_type=jnp.float32)
      + b1_ref[...])
  h = jnp.dot(h, w2_ref[...], preferred_element_type=jnp.float32) + b2_ref[...]
  u0_ref[...] = jnp.broadcast_to(dinv, (_N_PAD, _C)) * h
  c_ref[...] = jnp.broadcast_to((1.0 - _ALPHA) * dinv * dinv, (_N_PAD, _C))
  bh_ref[...] = _ALPHA * jnp.broadcast_to(dinv, (_N_PAD, _C)) * h
  sd_ref[...] = jnp.broadcast_to(
      jnp.where(mask, 1.0 / jnp.where(mask, dinv, 1.0), 0.0), (_N_PAD, _C))


_prep_call = pl.pallas_call(
    _prep_body,
    out_shape=[jax.ShapeDtypeStruct((_N_PAD, _C), jnp.float32)] * 4,
)


def _update_body(agg_ref, c_ref, bh_ref, u_ref):
  u_ref[...] = c_ref[...] * (agg_ref[0] + agg_ref[1]) + bh_ref[...]


_update_call = pl.pallas_call(
    _update_body,
    out_shape=jax.ShapeDtypeStruct((_N_PAD, _C), jnp.float32),
)


def _final_body(u_ref, sd_ref, out_ref):
  z = (u_ref[...] * sd_ref[...])[:_N]
  m = jnp.max(z, axis=1, keepdims=True)
  shifted = z - m
  out_ref[...] = shifted - jnp.log(
      jnp.sum(jnp.exp(shifted), axis=1, keepdims=True))


_final_call = pl.pallas_call(
    _final_body,
    out_shape=jax.ShapeDtypeStruct((_N, _C), jnp.float32),
)


def kernel(x, edge_index, W1, b1, W2, b2):
  e = edge_index.shape[1]
  e_full = e + _N
  n_slabs = -(-e_full // (_N_TILES * _SLAB * _CH))
  e_pad = _N_TILES * n_slabs * _SLAB * _CH

  src = edge_index[0]
  dst = edge_index[1]
  loop = jnp.arange(_N, dtype=jnp.int32)
  pad = jnp.full((e_pad - e_full,), _N, dtype=jnp.int32)
  src_w = jnp.concatenate([src, loop, pad]).reshape(
      _N_TILES, n_slabs, _SLAB * _CH)
  dst_w = jnp.concatenate([dst, loop, pad]).reshape(
      _N_TILES, n_slabs, _SLAB * _CH)

  x_pad = jnp.pad(x, ((0, _N_PAD - _N), (0, 0)))
  zeros_tile = jnp.zeros((_CH, _C), jnp.float32)
  ones_u = jnp.ones((_N_PAD, _C), jnp.float32)

  sweep = _make_edge_sweep(n_slabs)

  dega = sweep(ones_u, src_w, dst_w, zeros_tile)
  u, c, bh, sd = _prep_call(x_pad, W1, b1.reshape(1, _HID), W2,
                            b2.reshape(1, _C), dega)
  for _ in range(_K):
    agg = sweep(u, src_w, dst_w, zeros_tile)
    u = _update_call(agg, c, bh)
  return _final_call(u, sd)
